# baseline (speedup 1.0000x reference)
import functools

import jax
import jax.numpy as jnp
from jax import lax
from jax.experimental import pallas as pl
from jax.experimental.pallas import tpu as pltpu
from jax.experimental.pallas import tpu_sc as plsc

_NUM_WORKERS = 32
_NBUF = 3


def _plan(num_rows):
    assert num_rows % 8 == 0, num_rows
    granules = num_rows // 8
    lo = granules // _NUM_WORKERS
    nbig = granules - lo * _NUM_WORKERS
    small = lo * 8
    big = small + 8
    chunk, nchunks = None, None
    for c in range(min(small, 312), 0, -8):
        n = small // c
        if small % c == 0 and n % _NBUF == 0 and n >= 2 * _NBUF:
            chunk, nchunks = c, n
            break
    if chunk is None:
        chunk, nchunks = 8, small // 8
    return big, small, nbig, chunk, nchunks


@functools.lru_cache(maxsize=None)
def _make(num_rows, num_types, dim):
    big, small, nbig, chunk, nchunks = _plan(num_rows)
    mesh = plsc.VectorSubcoreMesh(core_axis_name="c", subcore_axis_name="s")
    tab_shards = 1
    for s in range(16, 0, -1):
        if num_types % s == 0 and (num_types // s) % 8 == 0:
            tab_shards = s
            break
    tab_rows = num_types // tab_shards

    @functools.partial(
        pl.kernel,
        mesh=mesh,
        out_type=jax.ShapeDtypeStruct((num_rows, dim), jnp.float32),
        scratch_types=(
            [pltpu.VMEM_SHARED((num_types, dim), jnp.float32),
             pltpu.VMEM((big,), jnp.int32)]
            + [pltpu.VMEM((chunk, dim), jnp.float32) for _ in range(_NBUF)]
            + [pltpu.VMEM((8, dim), jnp.float32)]
            + [pltpu.SemaphoreType.DMA for _ in range(2 * _NBUF)]
        ),
    )
    def gather_kernel(idx_hbm, table_hbm, out_hbm, table_sh, idx_v, *rest):
        rows = rest[:_NBUF]
        rows_t = rest[_NBUF]
        gsems = rest[_NBUF + 1:2 * _NBUF + 1]
        ssems = rest[2 * _NBUF + 1:]
        cid = lax.axis_index("c")
        sid = lax.axis_index("s")
        wid = sid * 2 + cid
        is_big = wid < nbig
        base = jnp.where(is_big, wid * big,
                         nbig * big + (wid - nbig) * small)
        base = pl.multiple_of(base, 8)

        pltpu.sync_copy(idx_hbm.at[pl.ds(base, small)],
                        idx_v.at[pl.ds(0, small)])
        if nbig:
            @pl.when(is_big)
            def _():
                off = pl.multiple_of(base + small, 8)
                pltpu.sync_copy(idx_hbm.at[pl.ds(off, 8)],
                                idx_v.at[pl.ds(small, 8)])

        @pl.when(sid < tab_shards)
        def _():
            shard = sid * tab_rows
            pltpu.sync_copy(table_hbm.at[pl.ds(shard, tab_rows), :],
                            table_sh.at[pl.ds(shard, tab_rows), :])
        plsc.subcore_barrier()

        def gather_start(j, b):
            off = pl.multiple_of(j * chunk, 8)
            return pltpu.async_copy(
                table_sh.at[idx_v.at[pl.ds(off, chunk)]], rows[b], gsems[b])

        def gather_drain(b):
            pltpu.make_async_copy(out_hbm.at[pl.ds(0, chunk), :], rows[b],
                                  gsems[b]).wait()

        def scatter_start(j, b):
            off = pl.multiple_of(base + j * chunk, 8)
            return pltpu.async_copy(rows[b],
                                    out_hbm.at[pl.ds(off, chunk), :],
                                    ssems[b])

        def scatter_drain(b):
            pltpu.make_async_copy(rows[b], out_hbm.at[pl.ds(0, chunk), :],
                                  ssems[b]).wait()

        gather_start(0, 0)
        gather_start(1, 1)
        gather_drain(0)
        gather_start(2, 2)
        scatter_start(0, 0)

        @pl.loop(1, nchunks - 2, step=_NBUF)
        def _(g):
            for b in range(_NBUF):
                j = g + b
                slot = (b + 1) % _NBUF
                nxt = (slot + 2) % _NBUF
                gather_drain(slot)
                scatter_drain(nxt)
                gather_start(j + 2, nxt)
                scatter_start(j, slot)

        for j in (nchunks - 2, nchunks - 1):
            slot = j % _NBUF
            gather_drain(slot)
            scatter_start(j, slot)

        if nbig:
            @pl.when(is_big)
            def _tail():
                pltpu.async_copy(
                    table_sh.at[idx_v.at[pl.ds(nchunks * chunk, 8)]],
                    rows_t, gsems[0]).wait()
                start = pl.multiple_of(base + nchunks * chunk, 8)
                pltpu.sync_copy(rows_t, out_hbm.at[pl.ds(start, 8), :])

        for b in range(_NBUF):
            scatter_drain(b)

    return gather_kernel


def kernel(node_types, type_embedding):
    (num_rows,) = node_types.shape
    num_types, dim = type_embedding.shape
    idx = node_types.astype(jnp.int32)
    table = type_embedding.astype(jnp.float32)
    return _make(num_rows, num_types, dim)(idx, table)
